# baseline (speedup 1.0000x reference)
import jax
import jax.numpy as jnp
from jax.experimental import pallas as pl
from jax.experimental.pallas import tpu as pltpu

_B = 8192
_IN = 16
_CTX = 384
_KD = 64
_NS = 512
_NSUB = 8
_HH = 128
_HO = 65
_BB = 1024


def _gelu(v):
    return 0.5 * v * (1.0 + jax.lax.erf(v * 0.7071067811865476))


def _fused(x_ref, ctx_ref, s2s_ref, keys_ref, qW1_ref, qb1_ref, qW2_ref,
           qb2_ref, w1_ref, hb1_ref, w2_ref, hb2_ref, temp_ref, out_ref):
    f32 = jnp.float32
    xb = x_ref[...]

    t = (jnp.dot(xb, qW1_ref[...], preferred_element_type=f32)
         + qb1_ref[...][None, :])
    t = _gelu(t)
    q = (jnp.dot(t, qW2_ref[...], preferred_element_type=f32)
         + qb2_ref[...][None, :])
    q = q * (1.0 / temp_ref[0])

    al = jax.lax.dot_general(q, keys_ref[...], (((1,), (1,)), ((), ())),
                             preferred_element_type=f32)
    al = al - jnp.max(al, axis=-1, keepdims=True)
    ea = jnp.exp(al)
    r = jnp.dot(ea, s2s_ref[...], preferred_element_type=f32)
    sl = r / jnp.sum(ea, axis=-1, keepdims=True)
    sl = sl - jnp.max(sl, axis=-1, keepdims=True)
    es = jnp.exp(sl)
    p = es / jnp.sum(es, axis=-1, keepdims=True)

    h = (jnp.dot(xb.astype(jnp.bfloat16), w1_ref[:_IN, :],
                 preferred_element_type=f32)
         + jnp.dot(ctx_ref[...].astype(jnp.bfloat16), w1_ref[_IN:, :],
                   preferred_element_type=f32)
         + hb1_ref[...])
    h = _gelu(h)

    eid = jax.lax.broadcasted_iota(jnp.int32, (_NSUB, _NSUB * _HH), 1) // _HH
    row = jax.lax.broadcasted_iota(jnp.int32, (_NSUB, _NSUB * _HH), 0)
    expand = (eid == row).astype(f32)
    pexp = jnp.dot(p, expand, preferred_element_type=f32)

    out = jnp.dot((h * pexp).astype(jnp.bfloat16), w2_ref[...],
                  preferred_element_type=f32)
    out = out + jnp.dot(p, hb2_ref[...], preferred_element_type=f32)
    out_ref[...] = out


def kernel(x, ctx, sys2sub, keys_p, qW1, qb1, qW2, qb2, hW1, hb1, hW2, hb2,
           temp):
    f32 = jnp.float32
    w1 = hW1.transpose(1, 0, 2).reshape(_IN + _CTX, _NSUB * _HH)
    w1 = w1.astype(jnp.bfloat16)
    w2 = hW2.reshape(_NSUB * _HH, _HO).astype(jnp.bfloat16)
    b1 = hb1.reshape(1, _NSUB * _HH)

    grid = (_B // _BB,)
    tok = lambda i: (i, 0)
    rep = lambda i: (0, 0)

    return pl.pallas_call(
        _fused,
        grid=grid,
        in_specs=[
            pl.BlockSpec((_BB, _IN), tok),
            pl.BlockSpec((_BB, _CTX), tok),
            pl.BlockSpec((_NS, _NSUB), rep),
            pl.BlockSpec((_NS, _KD), rep),
            pl.BlockSpec((_IN, _KD), rep),
            pl.BlockSpec((_KD,), lambda i: (0,)),
            pl.BlockSpec((_KD, _KD), rep),
            pl.BlockSpec((_KD,), lambda i: (0,)),
            pl.BlockSpec((_IN + _CTX, _NSUB * _HH), rep),
            pl.BlockSpec((1, _NSUB * _HH), rep),
            pl.BlockSpec((_NSUB * _HH, _HO), rep),
            pl.BlockSpec((_NSUB, _HO), rep),
            pl.BlockSpec(memory_space=pltpu.SMEM),
        ],
        out_specs=pl.BlockSpec((_BB, _HO), tok),
        out_shape=jax.ShapeDtypeStruct((_B, _HO), f32),
        compiler_params=pltpu.CompilerParams(
            dimension_semantics=("parallel",)),
    )(x, ctx, sys2sub, keys_p, qW1, qb1, qW2, qb2, w1, b1, w2, hb2,
      temp.reshape(1))


# speedup vs baseline: 1.0208x; 1.0208x over previous
import jax
import jax.numpy as jnp
from jax.experimental import pallas as pl
from jax.experimental.pallas import tpu as pltpu

_B = 8192
_IN = 16
_CTX = 384
_KD = 64
_NS = 512
_NSUB = 8
_HH = 128
_HO = 65
_BB = 1024


def _gelu(v):
    return 0.5 * v * (1.0 + jax.lax.erf(v * 0.7071067811865476))


def _fused(x_ref, ctx_ref, s2s_ref, keys_ref, qW1_ref, qb1_ref, qW2_ref,
           qb2_ref, w1_ref, hb1_ref, w2_ref, hb2_ref, temp_ref, out_ref):
    f32 = jnp.float32
    xb = x_ref[...]

    t = (jnp.dot(xb, qW1_ref[...], preferred_element_type=f32)
         + qb1_ref[...][None, :])
    t = _gelu(t)
    q = (jnp.dot(t, qW2_ref[...], preferred_element_type=f32)
         + qb2_ref[...][None, :])
    q = q * (1.0 / temp_ref[0])

    al = jax.lax.dot_general(q, keys_ref[...], (((1,), (1,)), ((), ())),
                             preferred_element_type=f32)
    al = al - jnp.max(al, axis=-1, keepdims=True)
    ea = jnp.exp(al)
    rz = jnp.dot(ea, s2s_ref[...], preferred_element_type=f32)
    sl = rz[:, :_NSUB] / rz[:, _NSUB:]
    sl = sl - jnp.max(sl, axis=-1, keepdims=True)
    es = jnp.exp(sl)
    p = es / jnp.sum(es, axis=-1, keepdims=True)

    h = (jnp.dot(xb, w1_ref[:_IN, :], preferred_element_type=f32)
         + jnp.dot(ctx_ref[...], w1_ref[_IN:, :], preferred_element_type=f32)
         + hb1_ref[...])
    h = _gelu(h)

    eid = jax.lax.broadcasted_iota(jnp.int32, (_NSUB, _NSUB * _HH), 1) // _HH
    row = jax.lax.broadcasted_iota(jnp.int32, (_NSUB, _NSUB * _HH), 0)
    expand = (eid == row).astype(f32)
    pexp = jnp.dot(p, expand, preferred_element_type=f32)

    out = jnp.dot(h * pexp, w2_ref[...], preferred_element_type=f32)
    out = out + jnp.dot(p, hb2_ref[...], preferred_element_type=f32)
    out_ref[...] = out


def kernel(x, ctx, sys2sub, keys_p, qW1, qb1, qW2, qb2, hW1, hb1, hW2, hb2,
           temp):
    f32 = jnp.float32
    w1 = hW1.transpose(1, 0, 2).reshape(_IN + _CTX, _NSUB * _HH)
    w2 = hW2.reshape(_NSUB * _HH, _HO)
    b1 = hb1.reshape(1, _NSUB * _HH)

    grid = (_B // _BB,)
    tok = lambda i: (i, 0)
    rep = lambda i: (0, 0)

    call = pl.pallas_call(
        _fused,
        grid=grid,
        in_specs=[
            pl.BlockSpec((_BB, _IN), tok),
            pl.BlockSpec((_BB, _CTX), tok),
            pl.BlockSpec((_NS, _NSUB + 1), rep),
            pl.BlockSpec((_NS, _KD), rep),
            pl.BlockSpec((_IN, _KD), rep),
            pl.BlockSpec((_KD,), lambda i: (0,)),
            pl.BlockSpec((_KD, _KD), rep),
            pl.BlockSpec((_KD,), lambda i: (0,)),
            pl.BlockSpec((_IN + _CTX, _NSUB * _HH), rep),
            pl.BlockSpec((1, _NSUB * _HH), rep),
            pl.BlockSpec((_NSUB * _HH, _HO), rep),
            pl.BlockSpec((_NSUB, _HO), rep),
            pl.BlockSpec(memory_space=pltpu.SMEM),
        ],
        out_specs=pl.BlockSpec((_BB, _HO), tok),
        out_shape=jax.ShapeDtypeStruct((_B, _HO), f32),
        compiler_params=pltpu.CompilerParams(
            dimension_semantics=("parallel",)),
    )
    s2s_aug = jnp.concatenate(
        [sys2sub, jnp.ones((_NS, 1), dtype=f32)], axis=1)
    return call(x, ctx, s2s_aug, keys_p, qW1, qb1, qW2, qb2, w1, b1, w2,
                hb2, temp.reshape(1))
